# baseline (speedup 1.0000x reference)
import functools

import jax
import jax.numpy as jnp
from jax import lax
from jax.experimental import pallas as pl
from jax.experimental.pallas import tpu as pltpu
from jax.experimental.pallas import tpu_sc as plsc

DIM = 768
NUM_EXPERTS = 8
INTER = 2048
TOP_K = 2
T = 16
LANES = 16
DCH = DIM // LANES


def _lane_perm(v, perm):
    return v.at[perm].get(mode="promise_in_bounds")


def _butterfly(v, op):
    lane = lax.iota(jnp.int32, LANES)
    for k in (8, 4, 2, 1):
        v = op(v, _lane_perm(v, lane ^ k))
    return v


def _sc_router_body(x_hbm, gate_hbm, out_hbm, xv, gv, cv):
    wid = lax.axis_index("s") + lax.axis_index("c") * 16

    @pl.when(wid < T)
    def _():
        t = wid
        pltpu.sync_copy(x_hbm.at[t], xv)
        pltpu.sync_copy(gate_hbm, gv)
        accs = [jnp.zeros((LANES,), jnp.float32) for _ in range(NUM_EXPERTS)]
        for j in range(DCH):
            xj = xv[pl.ds(j * LANES, LANES)]
            for e in range(NUM_EXPERTS):
                accs[e] = accs[e] + xj * gv[e, pl.ds(j * LANES, LANES)]
        lane = lax.iota(jnp.int32, LANES)
        validf = jnp.where(lane < NUM_EXPERTS, 1.0, 0.0)
        s = jnp.zeros((LANES,), jnp.float32)
        for e in range(NUM_EXPERTS):
            s = jnp.where(lane == e, _butterfly(accs[e], jnp.add), s)
        m = _butterfly(s * validf + (validf - 1.0) * 3.0e38, jnp.maximum)
        p = jnp.exp((s - m) * validf - 30.0 * (1.0 - validf)) * validf
        p = p / _butterfly(p, jnp.add)
        m1 = _butterfly(p, jnp.maximum)
        i1 = _butterfly(jnp.where(p == m1, lane, NUM_EXPERTS), jnp.minimum)
        oh1 = jnp.where(lane == i1, 1.0, 0.0)
        keep = (1.0 - oh1) * validf
        p_rest = p * keep - (1.0 - keep)
        m2 = _butterfly(p_rest, jnp.maximum)
        i2 = _butterfly(jnp.where(p_rest == m2, lane, NUM_EXPERTS),
                        jnp.minimum)
        oh2 = jnp.where(lane == i2, 1.0, 0.0)
        c = p * (oh1 + oh2)
        c = c / _butterfly(c, jnp.add)
        cv[...] = c
        pltpu.sync_copy(cv, out_hbm.at[t])


def _sc_router(x, gate_w):
    mesh = plsc.VectorSubcoreMesh(core_axis_name="c", subcore_axis_name="s",
                                  num_cores=1)
    fn = functools.partial(
        pl.kernel,
        mesh=mesh,
        out_type=jax.ShapeDtypeStruct((T, LANES), jnp.float32),
        scratch_types=[
            pltpu.VMEM((DIM,), jnp.float32),
            pltpu.VMEM((NUM_EXPERTS, DIM), jnp.float32),
            pltpu.VMEM((LANES,), jnp.float32),
        ],
    )(_sc_router_body)
    return fn(x, gate_w)


def _ffn_body(x_ref, w1_ref, w2_ref, w3_ref, out_ref):
    half = pl.program_id(1)
    xv = x_ref[...]
    dn = (((1,), (1,)), ((), ()))
    h1 = lax.dot_general(xv, w1_ref[0, 0], dn,
                         preferred_element_type=jnp.float32)
    h3 = lax.dot_general(xv, w3_ref[0, 0], dn,
                         preferred_element_type=jnp.float32)
    h = h1 * lax.logistic(h1) * h3
    part = lax.dot_general(h, w2_ref[0], dn,
                           preferred_element_type=jnp.float32)

    @pl.when(half == 0)
    def _():
        out_ref[0] = part

    @pl.when(half != 0)
    def _():
        out_ref[0] += part


def _combine_body(outs_ref, c_ref, out_ref):
    eidx = lax.broadcasted_iota(jnp.int32, (T, LANES), 1)
    acc = jnp.zeros((T, DIM), jnp.float32)
    for e in range(NUM_EXPERTS):
        col = jnp.sum(jnp.where(eidx == e, c_ref[...], 0.0),
                      axis=-1, keepdims=True)
        acc = acc + col * outs_ref[e]
    out_ref[...] = acc


def kernel(x, gate_w, w1, w2, w3):
    original_shape = x.shape
    xf = x.reshape(-1, DIM)
    nch = 4
    w1r = w1.reshape(NUM_EXPERTS, nch, INTER // nch, DIM)
    w3r = w3.reshape(NUM_EXPERTS, nch, INTER // nch, DIM)
    outs = pl.pallas_call(
        _ffn_body,
        grid=(NUM_EXPERTS, nch),
        in_specs=[
            pl.BlockSpec((T, DIM), lambda e, h: (0, 0)),
            pl.BlockSpec((1, 1, INTER // nch, DIM), lambda e, h: (e, h, 0, 0)),
            pl.BlockSpec((1, DIM, INTER // nch), lambda e, h: (e, 0, h)),
            pl.BlockSpec((1, 1, INTER // nch, DIM), lambda e, h: (e, h, 0, 0)),
        ],
        out_specs=pl.BlockSpec((1, T, DIM), lambda e, h: (e, 0, 0)),
        out_shape=jax.ShapeDtypeStruct((NUM_EXPERTS, T, DIM), jnp.float32),
    )(xf, w1r, w2, w3r)
    combine = _sc_router(xf, gate_w)
    out = pl.pallas_call(
        _combine_body,
        out_shape=jax.ShapeDtypeStruct((T, DIM), jnp.float32),
    )(outs, combine)
    return out.reshape(original_shape)


# speedup vs baseline: 1.1005x; 1.1005x over previous
import functools

import jax
import jax.numpy as jnp
from jax import lax
from jax.experimental import pallas as pl
from jax.experimental.pallas import tpu as pltpu
from jax.experimental.pallas import tpu_sc as plsc

DIM = 768
NUM_EXPERTS = 8
INTER = 2048
TOP_K = 2
T = 16
LANES = 16
DCH = DIM // LANES


def _lane_perm(v, perm):
    return v.at[perm].get(mode="promise_in_bounds")


def _butterfly(v, op):
    lane = lax.iota(jnp.int32, LANES)
    for k in (8, 4, 2, 1):
        v = op(v, _lane_perm(v, lane ^ k))
    return v


def _sc_router_body(x_hbm, gate_hbm, out_hbm, xv, gv, cv):
    wid = lax.axis_index("s") + lax.axis_index("c") * 16

    @pl.when(wid < T)
    def _():
        t = wid
        pltpu.sync_copy(x_hbm.at[t], xv)
        pltpu.sync_copy(gate_hbm, gv)
        accs = [jnp.zeros((LANES,), jnp.float32) for _ in range(NUM_EXPERTS)]
        for j in range(DCH):
            xj = xv[pl.ds(j * LANES, LANES)]
            for e in range(NUM_EXPERTS):
                accs[e] = accs[e] + xj * gv[e, pl.ds(j * LANES, LANES)]
        lane = lax.iota(jnp.int32, LANES)
        validf = jnp.where(lane < NUM_EXPERTS, 1.0, 0.0)
        s = jnp.zeros((LANES,), jnp.float32)
        for e in range(NUM_EXPERTS):
            s = jnp.where(lane == e, _butterfly(accs[e], jnp.add), s)
        m = _butterfly(s * validf + (validf - 1.0) * 3.0e38, jnp.maximum)
        p = jnp.exp((s - m) * validf - 30.0 * (1.0 - validf)) * validf
        p = p / _butterfly(p, jnp.add)
        m1 = _butterfly(p, jnp.maximum)
        i1 = _butterfly(jnp.where(p == m1, lane, NUM_EXPERTS), jnp.minimum)
        oh1 = jnp.where(lane == i1, 1.0, 0.0)
        keep = (1.0 - oh1) * validf
        p_rest = p * keep - (1.0 - keep)
        m2 = _butterfly(p_rest, jnp.maximum)
        i2 = _butterfly(jnp.where(p_rest == m2, lane, NUM_EXPERTS),
                        jnp.minimum)
        oh2 = jnp.where(lane == i2, 1.0, 0.0)
        c = p * (oh1 + oh2)
        c = c / _butterfly(c, jnp.add)
        cv[...] = c
        pltpu.sync_copy(cv, out_hbm.at[t])


def _sc_router(x, gate_w):
    mesh = plsc.VectorSubcoreMesh(core_axis_name="c", subcore_axis_name="s",
                                  num_cores=1)
    fn = functools.partial(
        pl.kernel,
        mesh=mesh,
        out_type=jax.ShapeDtypeStruct((T, LANES), jnp.float32),
        scratch_types=[
            pltpu.VMEM((DIM,), jnp.float32),
            pltpu.VMEM((NUM_EXPERTS, DIM), jnp.float32),
            pltpu.VMEM((LANES,), jnp.float32),
        ],
    )(_sc_router_body)
    return fn(x, gate_w)


def _ffn_body(x_ref, w1_ref, w2_ref, w3_ref, out_ref):
    half = pl.program_id(1)
    xv = x_ref[...]
    dn = (((1,), (1,)), ((), ()))
    h1 = lax.dot_general(xv, w1_ref[0, 0], dn,
                         preferred_element_type=jnp.float32)
    h3 = lax.dot_general(xv, w3_ref[0, 0], dn,
                         preferred_element_type=jnp.float32)
    h = h1 * lax.logistic(h1) * h3
    part = lax.dot_general(h, w2_ref[0], dn,
                           preferred_element_type=jnp.float32)

    @pl.when(half == 0)
    def _():
        out_ref[0] = part

    @pl.when(half == 1)
    def _():
        out_ref[0] += part


def _combine_body(outs_ref, c_ref, out_ref):
    eidx = lax.broadcasted_iota(jnp.int32, (T, LANES), 1)
    acc = jnp.zeros((T, DIM), jnp.float32)
    for e in range(NUM_EXPERTS):
        col = jnp.sum(jnp.where(eidx == e, c_ref[...], 0.0),
                      axis=-1, keepdims=True)
        acc = acc + col * outs_ref[e]
    out_ref[...] = acc


def kernel(x, gate_w, w1, w2, w3):
    original_shape = x.shape
    xf = x.reshape(-1, DIM)
    w1r = w1.reshape(NUM_EXPERTS, 2, INTER // 2, DIM)
    w3r = w3.reshape(NUM_EXPERTS, 2, INTER // 2, DIM)
    outs = pl.pallas_call(
        _ffn_body,
        grid=(NUM_EXPERTS, 2),
        in_specs=[
            pl.BlockSpec((T, DIM), lambda e, h: (0, 0)),
            pl.BlockSpec((1, 1, INTER // 2, DIM), lambda e, h: (e, h, 0, 0)),
            pl.BlockSpec((1, DIM, INTER // 2), lambda e, h: (e, 0, h)),
            pl.BlockSpec((1, 1, INTER // 2, DIM), lambda e, h: (e, h, 0, 0)),
        ],
        out_specs=pl.BlockSpec((1, T, DIM), lambda e, h: (e, 0, 0)),
        out_shape=jax.ShapeDtypeStruct((NUM_EXPERTS, T, DIM), jnp.float32),
    )(xf, w1r, w2, w3r)
    combine = _sc_router(xf, gate_w)
    out = pl.pallas_call(
        _combine_body,
        out_shape=jax.ShapeDtypeStruct((T, DIM), jnp.float32),
    )(outs, combine)
    return out.reshape(original_shape)
